# Initial kernel scaffold; baseline (speedup 1.0000x reference)
import jax
import jax.numpy as jnp
from jax.experimental import pallas as pl


def kernel(verts, edges, w0_weight, w0_bias, w1_weight, w1_bias):
    raise NotImplementedError("write your pallas kernel here")


# speedup vs baseline: 4.3643x; 4.3643x over previous
import functools

import jax
import jax.numpy as jnp
from jax import lax
from jax.experimental import pallas as pl
from jax.experimental.pallas import tpu as pltpu
from jax.experimental.pallas import tpu_sc as plsc

NC = 2
NS = 16
L = 16
NW = NC * NS

CHUNK = 128
ZROWS = 16


def _linear_body(x_ref, w_ref, b_ref, o_ref):
    o_ref[...] = lax.dot_general(
        x_ref[...], w_ref[...], (((1,), (1,)), ((), ())),
        preferred_element_type=jnp.float32) + b_ref[...]


def _combine_body(x_ref, w_ref, b_ref, p_ref, o_ref):
    o_ref[...] = (lax.dot_general(
        x_ref[...], w_ref[...], (((1,), (1,)), ((), ())),
        preferred_element_type=jnp.float32)
        + b_ref[...] + p_ref[0] + p_ref[1])


def _make_scatter(n_pad, n_acc, d, per_w):
    mesh = plsc.VectorSubcoreMesh(
        core_axis_name="c", subcore_axis_name="s",
        num_cores=NC, num_subcores=NS)

    @functools.partial(
        pl.kernel,
        mesh=mesh,
        out_type=jax.ShapeDtypeStruct((NC, n_acc, d), jnp.float32),
        scratch_types=[
            pltpu.VMEM((CHUNK,), jnp.int32),
            pltpu.VMEM((CHUNK,), jnp.int32),
            pltpu.VMEM((CHUNK, d), jnp.float32),
            pltpu.VMEM((ZROWS, d), jnp.float32),
            pltpu.VMEM_SHARED((n_acc, d), jnp.float32),
            pltpu.SemaphoreType.DMA,
        ],
    )
    def scatter(vw1_hbm, dst_hbm, src_hbm, out_hbm,
                dst_v, src_v, rows_v, zbuf, acc, sem):
        c = lax.axis_index("c")
        s = lax.axis_index("s")
        wid = s * NC + c

        for i in range(ZROWS):
            for k in range(d // L):
                zbuf[i, pl.ds(k * L, L)] = jnp.zeros((L,), jnp.float32)
        rows_per_tile = n_acc // NS

        def zbody(j, carry):
            pltpu.sync_copy(
                zbuf, acc.at[pl.ds(s * rows_per_tile + j * ZROWS, ZROWS)])
            return carry

        lax.fori_loop(0, rows_per_tile // ZROWS, zbody, 0)
        plsc.subcore_barrier()

        base0 = wid * per_w

        def body(j, carry):
            b = base0 + j * CHUNK
            pltpu.sync_copy(dst_hbm.at[pl.ds(b, CHUNK)], dst_v)
            pltpu.sync_copy(src_hbm.at[pl.ds(b, CHUNK)], src_v)
            pltpu.async_copy(vw1_hbm.at[src_v], rows_v, sem).wait()
            pltpu.sync_copy(rows_v, acc.at[dst_v], add=True)
            return carry

        lax.fori_loop(0, per_w // CHUNK, body, 0)
        plsc.subcore_barrier()

        out_rows = n_acc // NS
        pltpu.sync_copy(acc.at[pl.ds(s * out_rows, out_rows)],
                        out_hbm.at[c].at[pl.ds(s * out_rows, out_rows)])

    return scatter


def kernel(verts, edges, w0_weight, w0_bias, w1_weight, w1_bias):
    n, d_in = verts.shape
    d_out = w0_weight.shape[0]
    e = edges.shape[0]

    n_pad = ((n + 1 + NS * ZROWS - 1) // (NS * ZROWS)) * (NS * ZROWS)
    ed = 2 * e
    per_w = ((ed + NW * CHUNK - 1) // (NW * CHUNK)) * CHUNK
    tot = per_w * NW

    e32 = edges.astype(jnp.int32)
    fill = jnp.full((tot - ed,), n, jnp.int32)
    dst = jnp.concatenate([e32[:, 0], e32[:, 1], fill])
    src = jnp.concatenate([e32[:, 1], e32[:, 0], fill])

    verts_pad = jnp.zeros((n_pad, d_in), jnp.float32).at[:n].set(verts)
    b1 = w1_bias.reshape(1, d_out)
    b0 = w0_bias.reshape(1, d_out)

    blk = n_pad // 10 if n_pad % 10 == 0 else n_pad // 8
    vw1 = pl.pallas_call(
        _linear_body,
        grid=(n_pad // blk,),
        in_specs=[
            pl.BlockSpec((blk, d_in), lambda i: (i, 0)),
            pl.BlockSpec((d_out, d_in), lambda i: (0, 0)),
            pl.BlockSpec((1, d_out), lambda i: (0, 0)),
        ],
        out_specs=pl.BlockSpec((blk, d_out), lambda i: (i, 0)),
        out_shape=jax.ShapeDtypeStruct((n_pad, d_out), jnp.float32),
    )(verts_pad, w1_weight, b1)

    partials = _make_scatter(n_pad, n_pad, d_out, per_w)(vw1, dst, src)

    cblk = n // 10
    out = pl.pallas_call(
        _combine_body,
        grid=(n // cblk,),
        in_specs=[
            pl.BlockSpec((cblk, d_in), lambda i: (i, 0)),
            pl.BlockSpec((d_out, d_in), lambda i: (0, 0)),
            pl.BlockSpec((1, d_out), lambda i: (0, 0)),
            pl.BlockSpec((NC, cblk, d_out), lambda i: (0, i, 0)),
        ],
        out_specs=pl.BlockSpec((cblk, d_out), lambda i: (i, 0)),
        out_shape=jax.ShapeDtypeStruct((n, d_out), jnp.float32),
    )(verts, w0_weight, b0, partials)

    return out


# speedup vs baseline: 7.1553x; 1.6395x over previous
import functools

import jax
import jax.numpy as jnp
from jax import lax
from jax.experimental import pallas as pl
from jax.experimental.pallas import tpu as pltpu
from jax.experimental.pallas import tpu_sc as plsc

NC = 2
NS = 16
L = 16
NW = NC * NS

CHUNK = 128
ZROWS = 16


def _linear_body(x_ref, w_ref, b_ref, o_ref):
    o_ref[...] = lax.dot_general(
        x_ref[...], w_ref[...], (((1,), (1,)), ((), ())),
        preferred_element_type=jnp.float32) + b_ref[...]


def _combine_body(x_ref, w_ref, b_ref, p_ref, o_ref):
    o_ref[...] = (lax.dot_general(
        x_ref[...], w_ref[...], (((1,), (1,)), ((), ())),
        preferred_element_type=jnp.float32)
        + b_ref[...] + p_ref[0] + p_ref[1])


def _make_scatter(n_pad, n_acc, d, per_w):
    n_chunks = per_w // CHUNK
    mesh = plsc.VectorSubcoreMesh(
        core_axis_name="c", subcore_axis_name="s",
        num_cores=NC, num_subcores=NS)

    @functools.partial(
        pl.kernel,
        mesh=mesh,
        out_type=jax.ShapeDtypeStruct((NC, n_acc, d), jnp.float32),
        scratch_types=[
            pltpu.VMEM((2, CHUNK), jnp.int32),
            pltpu.VMEM((2, CHUNK), jnp.int32),
            pltpu.VMEM((CHUNK, d), jnp.float32),
            pltpu.VMEM((CHUNK, d), jnp.float32),
            pltpu.VMEM((ZROWS, d), jnp.float32),
            pltpu.VMEM_SHARED((n_acc, d), jnp.float32),
            pltpu.SemaphoreType.DMA,
            pltpu.SemaphoreType.DMA,
            pltpu.SemaphoreType.DMA,
            pltpu.SemaphoreType.DMA,
        ],
    )
    def scatter(vw1_hbm, pairs_hbm, out_hbm,
                idx0, idx1, rows0, rows1, zbuf, acc,
                sem_i0, sem_i1, sem_g0, sem_g1):
        c = lax.axis_index("c")
        s = lax.axis_index("s")
        wid = s * NC + c
        idx = (idx0, idx1)
        rows = (rows0, rows1)
        sem_i = (sem_i0, sem_i1)
        sem_g = (sem_g0, sem_g1)

        for i in range(ZROWS):
            for k in range(d // L):
                zbuf[i, pl.ds(k * L, L)] = jnp.zeros((L,), jnp.float32)
        rows_per_tile = n_acc // NS

        def zbody(j, carry):
            pltpu.sync_copy(
                zbuf, acc.at[pl.ds(s * rows_per_tile + j * ZROWS, ZROWS)])
            return carry

        lax.fori_loop(0, rows_per_tile // ZROWS, zbody, 0)
        plsc.subcore_barrier()

        base_c = wid * n_chunks

        def start_idx(jc, p):
            pltpu.async_copy(pairs_hbm.at[base_c + jc], idx[p], sem_i[p])

        def wait_idx(p):
            pltpu.make_async_copy(pairs_hbm.at[0], idx[p], sem_i[p]).wait()

        def start_gather(p):
            pltpu.async_copy(vw1_hbm.at[idx[p].at[1]], rows[p], sem_g[p])

        def wait_gather(p):
            pltpu.make_async_copy(
                vw1_hbm.at[idx[p].at[1]], rows[p], sem_g[p]).wait()

        def step(j, p):
            q = 1 - p
            wait_idx(p)
            start_gather(p)
            wait_gather(q)
            pltpu.sync_copy(rows[q], acc.at[idx[q].at[0]], add=True)
            start_idx(jnp.minimum(j + 1, n_chunks - 1), q)

        start_idx(0, 0)
        wait_idx(0)
        start_gather(0)
        start_idx(1, 1)

        def body(jo, carry):
            step(1 + 2 * jo, 1)
            step(2 + 2 * jo, 0)
            return carry

        lax.fori_loop(0, (n_chunks - 1) // 2, body, 0)
        wait_idx(1)
        wait_gather(0)
        pltpu.sync_copy(rows[0], acc.at[idx[0].at[0]], add=True)
        plsc.subcore_barrier()

        out_rows = n_acc // NS
        pltpu.sync_copy(acc.at[pl.ds(s * out_rows, out_rows)],
                        out_hbm.at[c].at[pl.ds(s * out_rows, out_rows)])

    return scatter


def kernel(verts, edges, w0_weight, w0_bias, w1_weight, w1_bias):
    n, d_in = verts.shape
    d_out = w0_weight.shape[0]
    e = edges.shape[0]

    n_pad = ((n + 1 + NS * ZROWS - 1) // (NS * ZROWS)) * (NS * ZROWS)
    ed = 2 * e
    per_w = ((ed + NW * CHUNK - 1) // (NW * CHUNK)) * CHUNK
    tot = per_w * NW

    e32 = edges.astype(jnp.int32)
    fill = jnp.full((tot - ed,), n, jnp.int32)
    dst = jnp.concatenate([e32[:, 0], e32[:, 1], fill])
    src = jnp.concatenate([e32[:, 1], e32[:, 0], fill])
    pairs = jnp.stack(
        [dst.reshape(-1, CHUNK), src.reshape(-1, CHUNK)], axis=1)

    verts_pad = jnp.zeros((n_pad, d_in), jnp.float32).at[:n].set(verts)
    b1 = w1_bias.reshape(1, d_out)
    b0 = w0_bias.reshape(1, d_out)

    blk = n_pad // 10 if n_pad % 10 == 0 else n_pad // 8
    vw1 = pl.pallas_call(
        _linear_body,
        grid=(n_pad // blk,),
        in_specs=[
            pl.BlockSpec((blk, d_in), lambda i: (i, 0)),
            pl.BlockSpec((d_out, d_in), lambda i: (0, 0)),
            pl.BlockSpec((1, d_out), lambda i: (0, 0)),
        ],
        out_specs=pl.BlockSpec((blk, d_out), lambda i: (i, 0)),
        out_shape=jax.ShapeDtypeStruct((n_pad, d_out), jnp.float32),
    )(verts_pad, w1_weight, b1)

    partials = _make_scatter(n_pad, n_pad, d_out, per_w)(vw1, pairs)

    cblk = n // 10
    out = pl.pallas_call(
        _combine_body,
        grid=(n // cblk,),
        in_specs=[
            pl.BlockSpec((cblk, d_in), lambda i: (i, 0)),
            pl.BlockSpec((d_out, d_in), lambda i: (0, 0)),
            pl.BlockSpec((1, d_out), lambda i: (0, 0)),
            pl.BlockSpec((NC, cblk, d_out), lambda i: (0, i, 0)),
        ],
        out_specs=pl.BlockSpec((cblk, d_out), lambda i: (i, 0)),
        out_shape=jax.ShapeDtypeStruct((n, d_out), jnp.float32),
    )(verts, w0_weight, b0, partials)

    return out


# speedup vs baseline: 7.1903x; 1.0049x over previous
import functools

import jax
import jax.numpy as jnp
from jax import lax
from jax.experimental import pallas as pl
from jax.experimental.pallas import tpu as pltpu
from jax.experimental.pallas import tpu_sc as plsc

NC = 2
NS = 16
L = 16
NW = NC * NS

CHUNK = 128
GPC = 1
ZROWS = 64


def _linear_body(x_ref, w_ref, b_ref, o_ref):
    o_ref[...] = lax.dot_general(
        x_ref[...], w_ref[...], (((1,), (1,)), ((), ())),
        preferred_element_type=jnp.float32) + b_ref[...]


def _combine_body(x_ref, w_ref, b_ref, p_ref, o_ref):
    o_ref[...] = (lax.dot_general(
        x_ref[...], w_ref[...], (((1,), (1,)), ((), ())),
        preferred_element_type=jnp.float32)
        + b_ref[...] + p_ref[0] + p_ref[1])


def _make_scatter(n_pad, n_acc, d, per_w):
    n_super = per_w // (GPC * CHUNK)
    mesh = plsc.VectorSubcoreMesh(
        core_axis_name="c", subcore_axis_name="s",
        num_cores=NC, num_subcores=NS)

    @functools.partial(
        pl.kernel,
        mesh=mesh,
        out_type=jax.ShapeDtypeStruct((NC, n_acc, d), jnp.float32),
        scratch_types=[
            pltpu.VMEM((GPC, 2, CHUNK), jnp.int32),
            pltpu.VMEM((GPC, 2, CHUNK), jnp.int32),
            pltpu.VMEM((GPC * CHUNK, d), jnp.float32),
            pltpu.VMEM((GPC * CHUNK, d), jnp.float32),
            pltpu.VMEM((ZROWS, d), jnp.float32),
            pltpu.VMEM_SHARED((n_acc, d), jnp.float32),
            pltpu.SemaphoreType.DMA,
            pltpu.SemaphoreType.DMA,
            pltpu.SemaphoreType.DMA,
            pltpu.SemaphoreType.DMA,
        ],
    )
    def scatter(vw1_hbm, pairs_hbm, out_hbm,
                idx0, idx1, rows0, rows1, zbuf, acc,
                sem_i0, sem_i1, sem_g0, sem_g1):
        c = lax.axis_index("c")
        s = lax.axis_index("s")
        wid = s * NC + c
        idx = (idx0, idx1)
        rows = (rows0, rows1)
        sem_i = (sem_i0, sem_i1)
        sem_g = (sem_g0, sem_g1)

        for i in range(ZROWS):
            for k in range(d // L):
                zbuf[i, pl.ds(k * L, L)] = jnp.zeros((L,), jnp.float32)
        rows_per_tile = n_acc // NS
        nz = rows_per_tile // ZROWS
        for t in range(nz):
            pltpu.async_copy(
                zbuf, acc.at[pl.ds(s * rows_per_tile + t * ZROWS, ZROWS)],
                sem_i0)
        for t in range(nz):
            pltpu.make_async_copy(
                zbuf, acc.at[pl.ds(s * rows_per_tile, ZROWS)], sem_i0).wait()
        plsc.subcore_barrier()

        base_c = wid * n_super * GPC

        def start_idx(jc, p):
            pltpu.async_copy(
                pairs_hbm.at[pl.ds(base_c + jc * GPC, GPC)], idx[p], sem_i[p])

        def wait_idx(p):
            pltpu.make_async_copy(
                pairs_hbm.at[pl.ds(0, GPC)], idx[p], sem_i[p]).wait()

        def start_gather(p):
            for u in range(GPC):
                pltpu.async_copy(
                    vw1_hbm.at[idx[p].at[u, 1]],
                    rows[p].at[pl.ds(u * CHUNK, CHUNK)], sem_g[p])

        def wait_gather(p):
            for u in range(GPC):
                pltpu.make_async_copy(
                    vw1_hbm.at[idx[p].at[u, 1]],
                    rows[p].at[pl.ds(u * CHUNK, CHUNK)], sem_g[p]).wait()

        def scatter_add(q):
            for u in range(GPC):
                pltpu.sync_copy(rows[q].at[pl.ds(u * CHUNK, CHUNK)],
                                acc.at[idx[q].at[u, 0]], add=True)

        def step(j, p):
            q = 1 - p
            wait_idx(p)
            start_gather(p)
            wait_gather(q)
            scatter_add(q)
            start_idx(jnp.minimum(j + 1, n_super - 1), q)

        start_idx(0, 0)
        wait_idx(0)
        start_gather(0)
        start_idx(1, 1)

        def body(jo, carry):
            step(1 + 2 * jo, 1)
            step(2 + 2 * jo, 0)
            return carry

        lax.fori_loop(0, (n_super - 1) // 2, body, 0)
        wait_idx(1)
        wait_gather(0)
        scatter_add(0)
        plsc.subcore_barrier()

        out_rows = n_acc // NS
        pltpu.sync_copy(acc.at[pl.ds(s * out_rows, out_rows)],
                        out_hbm.at[c].at[pl.ds(s * out_rows, out_rows)])

    return scatter


def kernel(verts, edges, w0_weight, w0_bias, w1_weight, w1_bias):
    n, d_in = verts.shape
    d_out = w0_weight.shape[0]
    e = edges.shape[0]

    n_pad = ((n + 1 + NS * ZROWS - 1) // (NS * ZROWS)) * (NS * ZROWS)
    ed = 2 * e
    sstep = GPC * CHUNK
    n_super = (ed + NW * sstep - 1) // (NW * sstep)
    if n_super % 2 == 0:
        n_super += 1
    per_w = n_super * sstep
    tot = per_w * NW

    e32 = edges.astype(jnp.int32)
    fill = jnp.full((tot - ed,), n, jnp.int32)
    dst = jnp.concatenate([e32[:, 0], e32[:, 1], fill])
    src = jnp.concatenate([e32[:, 1], e32[:, 0], fill])
    pairs = jnp.stack(
        [dst.reshape(-1, CHUNK), src.reshape(-1, CHUNK)], axis=1)

    verts_pad = jnp.zeros((n_pad, d_in), jnp.float32).at[:n].set(verts)
    b1 = w1_bias.reshape(1, d_out)
    b0 = w0_bias.reshape(1, d_out)

    blk = n_pad // 10 if n_pad % 10 == 0 else n_pad // 8
    vw1 = pl.pallas_call(
        _linear_body,
        grid=(n_pad // blk,),
        in_specs=[
            pl.BlockSpec((blk, d_in), lambda i: (i, 0)),
            pl.BlockSpec((d_out, d_in), lambda i: (0, 0)),
            pl.BlockSpec((1, d_out), lambda i: (0, 0)),
        ],
        out_specs=pl.BlockSpec((blk, d_out), lambda i: (i, 0)),
        out_shape=jax.ShapeDtypeStruct((n_pad, d_out), jnp.float32),
    )(verts_pad, w1_weight, b1)

    partials = _make_scatter(n_pad, n_pad, d_out, per_w)(vw1, pairs)

    cblk = n // 10
    out = pl.pallas_call(
        _combine_body,
        grid=(n // cblk,),
        in_specs=[
            pl.BlockSpec((cblk, d_in), lambda i: (i, 0)),
            pl.BlockSpec((d_out, d_in), lambda i: (0, 0)),
            pl.BlockSpec((1, d_out), lambda i: (0, 0)),
            pl.BlockSpec((NC, cblk, d_out), lambda i: (0, i, 0)),
        ],
        out_specs=pl.BlockSpec((cblk, d_out), lambda i: (i, 0)),
        out_shape=jax.ShapeDtypeStruct((n, d_out), jnp.float32),
    )(verts, w0_weight, b0, partials)

    return out
